# baseline (speedup 1.0000x reference)
import jax
import jax.numpy as jnp
from jax import lax
from jax.experimental import pallas as pl
from jax.experimental.pallas import tpu as pltpu

INPUT = 1024
HIDDEN = 4096
OUTPUT = 1024
E = 8
N = 2048
HCHUNK = 1024
NC = HIDDEN // HCHUNK
T = E * NC
NS = 8
NBLK = N // NS

_SQRT_HALF = 0.7071067811865476


def _fused_body(x_ref, wr_ref, br_ref, w1_ref, b1_ref, w2_ref, b2_ref,
                gamma_ref, beta_ref, out_ref, wcomb_s, xfirst_s, f_s):
    t = pl.program_id(0)
    e = jnp.minimum(t, T - 1) // NC

    @pl.when(t == 0)
    def _router():
        x = x_ref[...]
        logits = jnp.dot(x, wr_ref[...], preferred_element_type=jnp.float32)
        logits = logits + br_ref[...]

        iota_e = lax.broadcasted_iota(jnp.int32, (N, E), 1)
        m1 = jnp.max(logits, axis=-1, keepdims=True)
        a1 = jnp.min(jnp.where(logits == m1, iota_e, E), axis=-1, keepdims=True)
        masked = jnp.where(iota_e == a1, -jnp.inf, logits)
        m2 = jnp.max(masked, axis=-1, keepdims=True)
        a2 = jnp.min(jnp.where(masked == m2, iota_e, E), axis=-1, keepdims=True)

        r = jnp.exp(m2 - m1)
        denom = 1.0 + r
        p1 = 1.0 / denom
        p2 = r / denom

        sel1 = iota_e == a1
        sel2 = iota_e == a2
        wcomb_s[...] = jnp.where(sel1, p1, 0.0) + jnp.where(sel2, p2, 0.0)

        sel = sel1 | sel2
        iota_n = lax.broadcasted_iota(jnp.int32, (N, E), 0)
        fi = jnp.min(jnp.where(sel, iota_n, N), axis=0, keepdims=True)
        onehot = (iota_n == fi).astype(jnp.float32)
        xfirst_s[...] = lax.dot_general(
            onehot, x, (((0,), (0,)), ((), ())),
            preferred_element_type=jnp.float32)
        f_s[...] = b2_ref[:, 0, :]

    @pl.when(t < T)
    def _ffn():
        iota_row = lax.broadcasted_iota(jnp.int32, (1, E), 1)
        oh_e = (iota_row == e).astype(jnp.float32)
        xr = jnp.dot(oh_e, xfirst_s[...],
                     preferred_element_type=jnp.float32)

        h = jnp.dot(xr, w1_ref[0], preferred_element_type=jnp.float32)
        h = h + b1_ref[0]
        g = 0.5 * h * (1.0 + lax.erf(h * _SQRT_HALF))
        part = jnp.dot(g, w2_ref[0], preferred_element_type=jnp.float32)

        rmask = (lax.broadcasted_iota(jnp.int32, (E, 1), 0) == e)
        f_s[...] += rmask.astype(jnp.float32) * part

    @pl.when(t >= T)
    def _combine():
        blk = t - T
        rows = wcomb_s[pl.ds(blk * NBLK, NBLK), :]
        pre = jnp.dot(rows, f_s[...],
                      preferred_element_type=jnp.float32)
        mean = jnp.mean(pre, axis=-1, keepdims=True)
        d = pre - mean
        var = jnp.mean(d * d, axis=-1, keepdims=True)
        inv = lax.rsqrt(var + 1e-5)
        out_ref[...] = d * inv * gamma_ref[...] + beta_ref[...]


@jax.jit
def kernel(x, Wr, br, W1, b1, W2, b2, gamma, beta):
    Bc, S, D = x.shape
    xf = x.reshape(Bc * S, D)

    def wmap(t):
        tc = jnp.minimum(t, T - 1)
        return tc // NC, tc % NC

    out = pl.pallas_call(
        _fused_body,
        grid=(T + NS,),
        in_specs=[
            pl.BlockSpec((N, INPUT), lambda t: (0, 0)),
            pl.BlockSpec((INPUT, E), lambda t: (0, 0)),
            pl.BlockSpec((1, E), lambda t: (0, 0)),
            pl.BlockSpec((1, INPUT, HCHUNK),
                         lambda t: (wmap(t)[0], 0, wmap(t)[1])),
            pl.BlockSpec((1, 1, HCHUNK),
                         lambda t: (wmap(t)[0], 0, wmap(t)[1])),
            pl.BlockSpec((1, HCHUNK, OUTPUT),
                         lambda t: (wmap(t)[0], wmap(t)[1], 0)),
            pl.BlockSpec((E, 1, OUTPUT), lambda t: (0, 0, 0)),
            pl.BlockSpec((1, OUTPUT), lambda t: (0, 0)),
            pl.BlockSpec((1, OUTPUT), lambda t: (0, 0)),
        ],
        out_specs=pl.BlockSpec(
            (NBLK, OUTPUT),
            lambda t: (jnp.clip(t - T, 0, NS - 1), 0)),
        out_shape=jax.ShapeDtypeStruct((N, OUTPUT), jnp.float32),
        scratch_shapes=[
            pltpu.VMEM((N, E), jnp.float32),
            pltpu.VMEM((E, INPUT), jnp.float32),
            pltpu.VMEM((E, OUTPUT), jnp.float32),
        ],
    )(xf, Wr, br.reshape(1, E), W1, b1.reshape(E, 1, HIDDEN),
      W2, b2.reshape(E, 1, OUTPUT), gamma.reshape(1, OUTPUT),
      beta.reshape(1, OUTPUT))

    return out.reshape(Bc, S, OUTPUT)


# speedup vs baseline: 1.0148x; 1.0148x over previous
import jax
import jax.numpy as jnp
from jax import lax
from jax.experimental import pallas as pl
from jax.experimental.pallas import tpu as pltpu

INPUT = 1024
HIDDEN = 4096
OUTPUT = 1024
E = 8
N = 2048
HCHUNK = 1024
NC = HIDDEN // HCHUNK
T = E * NC
NS = 4
NBLK = N // NS

_SQRT_HALF = 0.7071067811865476


def _fused_body(x_ref, wr_ref, br_ref, w1_ref, b1_ref, w2_ref, b2_ref,
                gamma_ref, beta_ref, out_ref, wcomb_s, xfirst_s, f_s):
    t = pl.program_id(0)
    e = jnp.minimum(t, T - 1) // NC

    @pl.when(t == 0)
    def _router():
        x = x_ref[...]
        logits = jnp.dot(x, wr_ref[...], preferred_element_type=jnp.float32)
        logits = logits + br_ref[...]

        iota_e = lax.broadcasted_iota(jnp.int32, (N, E), 1)
        m1 = jnp.max(logits, axis=-1, keepdims=True)
        a1 = jnp.min(jnp.where(logits == m1, iota_e, E), axis=-1, keepdims=True)
        masked = jnp.where(iota_e == a1, -jnp.inf, logits)
        m2 = jnp.max(masked, axis=-1, keepdims=True)
        a2 = jnp.min(jnp.where(masked == m2, iota_e, E), axis=-1, keepdims=True)

        r = jnp.exp(m2 - m1)
        denom = 1.0 + r
        p1 = 1.0 / denom
        p2 = r / denom

        sel1 = iota_e == a1
        sel2 = iota_e == a2
        wcomb_s[...] = jnp.where(sel1, p1, 0.0) + jnp.where(sel2, p2, 0.0)

        sel = sel1 | sel2
        iota_n = lax.broadcasted_iota(jnp.int32, (N, E), 0)
        fi = jnp.min(jnp.where(sel, iota_n, N), axis=0, keepdims=True)
        onehot = (iota_n == fi).astype(jnp.float32)
        xfirst_s[...] = lax.dot_general(
            onehot, x, (((0,), (0,)), ((), ())),
            preferred_element_type=jnp.float32)
        f_s[...] = b2_ref[:, 0, :]

    @pl.when(t < T)
    def _ffn():
        iota_row = lax.broadcasted_iota(jnp.int32, (1, E), 1)
        oh_e = (iota_row == e).astype(jnp.float32)
        xr = jnp.dot(oh_e, xfirst_s[...],
                     preferred_element_type=jnp.float32)

        h = jnp.dot(xr, w1_ref[0], preferred_element_type=jnp.float32)
        h = h + b1_ref[0]
        g = 0.5 * h * (1.0 + lax.erf(h * _SQRT_HALF))
        part = jnp.dot(g, w2_ref[0], preferred_element_type=jnp.float32)

        rmask = (lax.broadcasted_iota(jnp.int32, (E, 1), 0) == e)
        f_s[...] += rmask.astype(jnp.float32) * part

    @pl.when(t >= T)
    def _combine():
        blk = t - T
        rows = wcomb_s[pl.ds(blk * NBLK, NBLK), :]
        pre = jnp.dot(rows, f_s[...],
                      preferred_element_type=jnp.float32)
        mean = jnp.mean(pre, axis=-1, keepdims=True)
        d = pre - mean
        var = jnp.mean(d * d, axis=-1, keepdims=True)
        inv = lax.rsqrt(var + 1e-5)
        out_ref[...] = d * inv * gamma_ref[...] + beta_ref[...]


@jax.jit
def kernel(x, Wr, br, W1, b1, W2, b2, gamma, beta):
    Bc, S, D = x.shape
    xf = x.reshape(Bc * S, D)

    def wmap(t):
        tc = jnp.minimum(t, T - 1)
        return tc // NC, tc % NC

    out = pl.pallas_call(
        _fused_body,
        grid=(T + NS,),
        in_specs=[
            pl.BlockSpec((N, INPUT), lambda t: (0, 0)),
            pl.BlockSpec((INPUT, E), lambda t: (0, 0)),
            pl.BlockSpec((1, E), lambda t: (0, 0)),
            pl.BlockSpec((1, INPUT, HCHUNK),
                         lambda t: (wmap(t)[0], 0, wmap(t)[1])),
            pl.BlockSpec((1, 1, HCHUNK),
                         lambda t: (wmap(t)[0], 0, wmap(t)[1])),
            pl.BlockSpec((1, HCHUNK, OUTPUT),
                         lambda t: (wmap(t)[0], wmap(t)[1], 0)),
            pl.BlockSpec((E, 1, OUTPUT), lambda t: (0, 0, 0)),
            pl.BlockSpec((1, OUTPUT), lambda t: (0, 0)),
            pl.BlockSpec((1, OUTPUT), lambda t: (0, 0)),
        ],
        out_specs=pl.BlockSpec(
            (NBLK, OUTPUT),
            lambda t: (jnp.clip(t - T, 0, NS - 1), 0)),
        out_shape=jax.ShapeDtypeStruct((N, OUTPUT), jnp.float32),
        scratch_shapes=[
            pltpu.VMEM((N, E), jnp.float32),
            pltpu.VMEM((E, INPUT), jnp.float32),
            pltpu.VMEM((E, OUTPUT), jnp.float32),
        ],
    )(xf, Wr, br.reshape(1, E), W1, b1.reshape(E, 1, HIDDEN),
      W2, b2.reshape(E, 1, OUTPUT), gamma.reshape(1, OUTPUT),
      beta.reshape(1, OUTPUT))

    return out.reshape(Bc, S, OUTPUT)
